# Initial kernel scaffold; baseline (speedup 1.0000x reference)
import jax
import jax.numpy as jnp
from jax.experimental import pallas as pl


def kernel(x, edge_index, W1, b1, bn1_gamma, bn1_beta, W2, b2, bn2_gamma, bn2_beta):
    raise NotImplementedError("write your pallas kernel here")


# speedup vs baseline: 6.9656x; 6.9656x over previous
import functools
import math

import jax
import jax.numpy as jnp
from jax import lax
from jax.experimental import pallas as pl
from jax.experimental.pallas import tpu as pltpu
from jax.experimental.pallas import tpu_sc as plsc

N = 10000
E = 320000
D = 128
EPS = 1e-5

NC = 2
NS = 16
NW = NC * NS
C = 128
CPW = 80
TOT_CH = NW * CPW
E_PAD = TOT_CH * C
N_ACC = 10240
TRASH = 10100
RPT = N_ACC // NS
ZR = 64
DEGW = 128

def _zero_rows(zref, rows, width):
    def row(r, _):
        def col(cc, __):
            zref[r, pl.ds(cc * 16, 16)] = jnp.zeros((16,), jnp.float32)
            return __
        return lax.fori_loop(0, width // 16, col, _)
    lax.fori_loop(0, rows, row, None)


def _deg_sc_body(dst_hbm, out_hbm, idx_v, ones_v, z_v, acc_sh):
    c = lax.axis_index("c")
    s = lax.axis_index("s")
    w = c * NS + s
    _zero_rows(z_v, ZR, DEGW)
    def orow(r, _):
        def ocol(cc, __):
            ones_v[r, pl.ds(cc * 16, 16)] = jnp.full((16,), 1.0, jnp.float32)
            return __
        return lax.fori_loop(0, DEGW // 16, ocol, _)
    lax.fori_loop(0, C, orow, None)
    def zcopy(k, _):
        pltpu.sync_copy(z_v, acc_sh.at[pl.ds(s * RPT + k * ZR, ZR)])
        return _
    lax.fori_loop(0, RPT // ZR, zcopy, None)
    pltpu.sync_copy(dst_hbm.at[pl.ds(w * CPW, CPW)], idx_v)
    plsc.subcore_barrier()
    def step(j, _):
        pltpu.sync_copy(ones_v, acc_sh.at[idx_v.at[j]], add=True)
        return _
    lax.fori_loop(0, CPW, step, None)
    plsc.subcore_barrier()
    pltpu.sync_copy(acc_sh.at[pl.ds(s * RPT, RPT)],
                    out_hbm.at[c, pl.ds(s * RPT, RPT)])


def _agg_sc_body(hws_hbm, src_hbm, dst_hbm, out_hbm,
                 src_v, dst_v, gbuf, z_v, acc_sh, sem):
    c = lax.axis_index("c")
    s = lax.axis_index("s")
    w = c * NS + s
    _zero_rows(z_v, ZR, D)
    def zcopy(k, _):
        pltpu.sync_copy(z_v, acc_sh.at[pl.ds(s * RPT + k * ZR, ZR)])
        return _
    lax.fori_loop(0, RPT // ZR, zcopy, None)
    pltpu.sync_copy(src_hbm.at[pl.ds(w * CPW, CPW)], src_v)
    pltpu.sync_copy(dst_hbm.at[pl.ds(w * CPW, CPW)], dst_v)
    plsc.subcore_barrier()
    def step(j, _):
        pltpu.async_copy(hws_hbm.at[src_v.at[j]], gbuf, sem).wait()
        pltpu.sync_copy(gbuf, acc_sh.at[dst_v.at[j]], add=True)
        return _
    lax.fori_loop(0, CPW, step, None)
    plsc.subcore_barrier()
    pltpu.sync_copy(acc_sh.at[pl.ds(s * RPT, RPT)],
                    out_hbm.at[c, pl.ds(s * RPT, RPT)])


def _mm_body(x_ref, w_ref, o_ref):
    o_ref[...] = jnp.dot(x_ref[...], w_ref[...],
                         preferred_element_type=jnp.float32)


def _scale_body(hw_ref, d0_ref, d1_ref, dinv_ref, hws_ref):
    deg = d0_ref[...] + d1_ref[...] + 1.0
    dinv = lax.rsqrt(deg)
    dinv_ref[...] = dinv
    hws_ref[...] = hw_ref[...] * dinv


def _bn_relu(t, gamma, beta):
    mu = jnp.mean(t, axis=0, keepdims=True)
    var = jnp.mean((t - mu) ** 2, axis=0, keepdims=True)
    y = gamma * (t - mu) * lax.rsqrt(var + EPS) + beta
    return jnp.maximum(y, 0.0)


def _mid_body(a0_ref, a1_ref, hws_ref, dinv_ref, b_ref, g_ref, be_ref,
              w2_ref, hws2_ref):
    t = dinv_ref[...] * (a0_ref[...] + a1_ref[...] + hws_ref[...]) + b_ref[...]
    h1 = _bn_relu(t, g_ref[...], be_ref[...])
    hw2 = jnp.dot(h1, w2_ref[...], preferred_element_type=jnp.float32)
    hws2_ref[...] = hw2 * dinv_ref[...]


def _final_body(a0_ref, a1_ref, hws_ref, dinv_ref, b_ref, g_ref, be_ref,
                x_ref, o_ref):
    t = dinv_ref[...] * (a0_ref[...] + a1_ref[...] + hws_ref[...]) + b_ref[...]
    y = _bn_relu(t, g_ref[...], be_ref[...])
    o_ref[...] = (y + x_ref[...]) * (1.0 / math.sqrt(2.0))


def _tc(body, out_shapes, *args):
    return pl.pallas_call(body, out_shape=out_shapes)(*args)


@functools.cache
def _sc_kernels():
    mesh = plsc.VectorSubcoreMesh(core_axis_name="c", subcore_axis_name="s",
                                  num_cores=NC, num_subcores=NS)
    deg = pl.kernel(
        _deg_sc_body,
        out_type=jax.ShapeDtypeStruct((NC, N_ACC, DEGW), jnp.float32),
        mesh=mesh,
        scratch_types=[
            pltpu.VMEM((CPW, C), jnp.int32),
            pltpu.VMEM((C, DEGW), jnp.float32),
            pltpu.VMEM((ZR, DEGW), jnp.float32),
            pltpu.VMEM_SHARED((N_ACC, DEGW), jnp.float32),
        ],
    )
    agg = pl.kernel(
        _agg_sc_body,
        out_type=jax.ShapeDtypeStruct((NC, N_ACC, D), jnp.float32),
        mesh=mesh,
        scratch_types=[
            pltpu.VMEM((CPW, C), jnp.int32),
            pltpu.VMEM((CPW, C), jnp.int32),
            pltpu.VMEM((C, D), jnp.float32),
            pltpu.VMEM((ZR, D), jnp.float32),
            pltpu.VMEM_SHARED((N_ACC, D), jnp.float32),
            pltpu.SemaphoreType.DMA,
        ],
    )
    return deg, agg


def kernel(x, edge_index, W1, b1, bn1_gamma, bn1_beta,
           W2, b2, bn2_gamma, bn2_beta):
    src = edge_index[0].astype(jnp.int32)
    dst = edge_index[1].astype(jnp.int32)
    pad = E_PAD - E
    src2d = jnp.concatenate(
        [src, jnp.zeros((pad,), jnp.int32)]).reshape(TOT_CH, C)
    dst2d = jnp.concatenate(
        [dst, jnp.full((pad,), TRASH, jnp.int32)]).reshape(TOT_CH, C)

    _deg_sc, _agg_sc = _sc_kernels()
    degp = _deg_sc(dst2d)
    hw1 = _tc(_mm_body, jax.ShapeDtypeStruct((N, D), jnp.float32), x, W1)

    d0 = degp[0, :N, 0:1]
    d1 = degp[1, :N, 0:1]
    dinv, hws1 = _tc(
        _scale_body,
        (jax.ShapeDtypeStruct((N, 1), jnp.float32),
         jax.ShapeDtypeStruct((N, D), jnp.float32)),
        hw1, d0, d1)

    b1r = b1.reshape(1, D)
    g1r = bn1_gamma.reshape(1, D)
    be1r = bn1_beta.reshape(1, D)
    b2r = b2.reshape(1, D)
    g2r = bn2_gamma.reshape(1, D)
    be2r = bn2_beta.reshape(1, D)

    agg1 = _agg_sc(hws1, src2d, dst2d)
    hws2 = _tc(
        _mid_body, jax.ShapeDtypeStruct((N, D), jnp.float32),
        agg1[0, :N], agg1[1, :N], hws1, dinv, b1r, g1r, be1r, W2)

    agg2 = _agg_sc(hws2, src2d, dst2d)
    out = _tc(
        _final_body, jax.ShapeDtypeStruct((N, D), jnp.float32),
        agg2[0, :N], agg2[1, :N], hws2, dinv, b2r, g2r, be2r, x)
    return out


# speedup vs baseline: 7.3726x; 1.0584x over previous
import functools
import math

import jax
import jax.numpy as jnp
from jax import lax
from jax.experimental import pallas as pl
from jax.experimental.pallas import tpu as pltpu
from jax.experimental.pallas import tpu_sc as plsc

N = 10000
E = 320000
D = 128
EPS = 1e-5

NC = 2
NS = 16
NW = NC * NS
C = 128
CPW = 80
TOT_CH = NW * CPW
E_PAD = TOT_CH * C
N_ACC = 10240
TRASH = 10100
RPT = N_ACC // NS
ZR = 64
DEGW = 128

def _zero_rows(zref, rows, width):
    def row(r, _):
        def col(cc, __):
            zref[r, pl.ds(cc * 16, 16)] = jnp.zeros((16,), jnp.float32)
            return __
        return lax.fori_loop(0, width // 16, col, _)
    lax.fori_loop(0, rows, row, None)


def _deg_sc_body(dst_hbm, out_hbm, idx_v, ones_v, z_v, acc_sh):
    c = lax.axis_index("c")
    s = lax.axis_index("s")
    w = c * NS + s
    _zero_rows(z_v, ZR, DEGW)
    def orow(r, _):
        def ocol(cc, __):
            ones_v[r, pl.ds(cc * 16, 16)] = jnp.full((16,), 1.0, jnp.float32)
            return __
        return lax.fori_loop(0, DEGW // 16, ocol, _)
    lax.fori_loop(0, C, orow, None)
    def zcopy(k, _):
        pltpu.sync_copy(z_v, acc_sh.at[pl.ds(s * RPT + k * ZR, ZR)])
        return _
    lax.fori_loop(0, RPT // ZR, zcopy, None)
    pltpu.sync_copy(dst_hbm.at[pl.ds(w * CPW, CPW)], idx_v)
    plsc.subcore_barrier()
    def step(j, _):
        pltpu.sync_copy(ones_v, acc_sh.at[idx_v.at[j]], add=True)
        return _
    lax.fori_loop(0, CPW, step, None)
    plsc.subcore_barrier()
    pltpu.sync_copy(acc_sh.at[pl.ds(s * RPT, RPT)],
                    out_hbm.at[c, pl.ds(s * RPT, RPT)])


BLK = 8


def _agg_sc_body(hws_hbm, src_hbm, dst_hbm, out_hbm,
                 sblk, dblk, gbuf0, gbuf1, z_v, acc_sh,
                 gsem0, gsem1, ssem0, ssem1, isem):
    c = lax.axis_index("c")
    s = lax.axis_index("s")
    w = c * NS + s
    _zero_rows(z_v, ZR, D)
    def zcopy(k, _):
        pltpu.sync_copy(z_v, acc_sh.at[pl.ds(s * RPT + k * ZR, ZR)])
        return _
    lax.fori_loop(0, RPT // ZR, zcopy, None)
    plsc.subcore_barrier()

    gb = (gbuf0, gbuf1)
    gsem = (gsem0, gsem1)
    ssem = (ssem0, ssem1)

    @pl.loop(0, CPW, step=BLK)
    def _block(jj):
        base = w * CPW + jj
        ic0 = pltpu.async_copy(src_hbm.at[pl.ds(base, BLK)], sblk, isem)
        ic1 = pltpu.async_copy(dst_hbm.at[pl.ds(base, BLK)], dblk, isem)
        ic0.wait()
        ic1.wait()
        prev_scat = [None, None]
        cur = pltpu.async_copy(hws_hbm.at[sblk.at[0]], gb[0], gsem[0])
        for i in range(BLK):
            p = i & 1
            q = 1 - p
            cur.wait()
            if i < BLK - 1:
                if prev_scat[q] is not None:
                    prev_scat[q].wait()
                cur = pltpu.async_copy(hws_hbm.at[sblk.at[i + 1]],
                                       gb[q], gsem[q])
            prev_scat[p] = pltpu.async_copy(gb[p], acc_sh.at[dblk.at[i]],
                                            ssem[p], add=True)
        prev_scat[0].wait()
        prev_scat[1].wait()

    plsc.subcore_barrier()
    pltpu.sync_copy(acc_sh.at[pl.ds(s * RPT, RPT)],
                    out_hbm.at[c, pl.ds(s * RPT, RPT)])


def _mm_body(x_ref, w_ref, o_ref):
    o_ref[...] = jnp.dot(x_ref[...], w_ref[...],
                         preferred_element_type=jnp.float32)


def _scale_body(hw_ref, d0_ref, d1_ref, dinv_ref, hws_ref):
    deg = d0_ref[...] + d1_ref[...] + 1.0
    dinv = lax.rsqrt(deg)
    dinv_ref[...] = dinv
    hws_ref[...] = hw_ref[...] * dinv


def _bn_relu(t, gamma, beta):
    mu = jnp.mean(t, axis=0, keepdims=True)
    var = jnp.mean((t - mu) ** 2, axis=0, keepdims=True)
    y = gamma * (t - mu) * lax.rsqrt(var + EPS) + beta
    return jnp.maximum(y, 0.0)


def _mid_body(a0_ref, a1_ref, hws_ref, dinv_ref, b_ref, g_ref, be_ref,
              w2_ref, hws2_ref):
    t = dinv_ref[...] * (a0_ref[...] + a1_ref[...] + hws_ref[...]) + b_ref[...]
    h1 = _bn_relu(t, g_ref[...], be_ref[...])
    hw2 = jnp.dot(h1, w2_ref[...], preferred_element_type=jnp.float32)
    hws2_ref[...] = hw2 * dinv_ref[...]


def _final_body(a0_ref, a1_ref, hws_ref, dinv_ref, b_ref, g_ref, be_ref,
                x_ref, o_ref):
    t = dinv_ref[...] * (a0_ref[...] + a1_ref[...] + hws_ref[...]) + b_ref[...]
    y = _bn_relu(t, g_ref[...], be_ref[...])
    o_ref[...] = (y + x_ref[...]) * (1.0 / math.sqrt(2.0))


def _tc(body, out_shapes, *args):
    return pl.pallas_call(body, out_shape=out_shapes)(*args)


@functools.cache
def _sc_kernels():
    mesh = plsc.VectorSubcoreMesh(core_axis_name="c", subcore_axis_name="s",
                                  num_cores=NC, num_subcores=NS)
    deg = pl.kernel(
        _deg_sc_body,
        out_type=jax.ShapeDtypeStruct((NC, N_ACC, DEGW), jnp.float32),
        mesh=mesh,
        scratch_types=[
            pltpu.VMEM((CPW, C), jnp.int32),
            pltpu.VMEM((C, DEGW), jnp.float32),
            pltpu.VMEM((ZR, DEGW), jnp.float32),
            pltpu.VMEM_SHARED((N_ACC, DEGW), jnp.float32),
        ],
    )
    agg = pl.kernel(
        _agg_sc_body,
        out_type=jax.ShapeDtypeStruct((NC, N_ACC, D), jnp.float32),
        mesh=mesh,
        scratch_types=[
            pltpu.VMEM((BLK, C), jnp.int32),
            pltpu.VMEM((BLK, C), jnp.int32),
            pltpu.VMEM((C, D), jnp.float32),
            pltpu.VMEM((C, D), jnp.float32),
            pltpu.VMEM((ZR, D), jnp.float32),
            pltpu.VMEM_SHARED((N_ACC, D), jnp.float32),
            pltpu.SemaphoreType.DMA,
            pltpu.SemaphoreType.DMA,
            pltpu.SemaphoreType.DMA,
            pltpu.SemaphoreType.DMA,
            pltpu.SemaphoreType.DMA,
        ],
    )
    return deg, agg


def kernel(x, edge_index, W1, b1, bn1_gamma, bn1_beta,
           W2, b2, bn2_gamma, bn2_beta):
    src = edge_index[0].astype(jnp.int32)
    dst = edge_index[1].astype(jnp.int32)
    pad = E_PAD - E
    src2d = jnp.concatenate(
        [src, jnp.zeros((pad,), jnp.int32)]).reshape(TOT_CH, C)
    dst2d = jnp.concatenate(
        [dst, jnp.full((pad,), TRASH, jnp.int32)]).reshape(TOT_CH, C)

    _deg_sc, _agg_sc = _sc_kernels()
    degp = _deg_sc(dst2d)
    hw1 = _tc(_mm_body, jax.ShapeDtypeStruct((N, D), jnp.float32), x, W1)

    d0 = degp[0, :N, 0:1]
    d1 = degp[1, :N, 0:1]
    dinv, hws1 = _tc(
        _scale_body,
        (jax.ShapeDtypeStruct((N, 1), jnp.float32),
         jax.ShapeDtypeStruct((N, D), jnp.float32)),
        hw1, d0, d1)

    b1r = b1.reshape(1, D)
    g1r = bn1_gamma.reshape(1, D)
    be1r = bn1_beta.reshape(1, D)
    b2r = b2.reshape(1, D)
    g2r = bn2_gamma.reshape(1, D)
    be2r = bn2_beta.reshape(1, D)

    agg1 = _agg_sc(hws1, src2d, dst2d)
    hws2 = _tc(
        _mid_body, jax.ShapeDtypeStruct((N, D), jnp.float32),
        agg1[0, :N], agg1[1, :N], hws1, dinv, b1r, g1r, be1r, W2)

    agg2 = _agg_sc(hws2, src2d, dst2d)
    out = _tc(
        _final_body, jax.ShapeDtypeStruct((N, D), jnp.float32),
        agg2[0, :N], agg2[1, :N], hws2, dinv, b2r, g2r, be2r, x)
    return out


# speedup vs baseline: 7.6377x; 1.0359x over previous
import functools
import math

import jax
import jax.numpy as jnp
from jax import lax
from jax.experimental import pallas as pl
from jax.experimental.pallas import tpu as pltpu
from jax.experimental.pallas import tpu_sc as plsc

N = 10000
E = 320000
D = 128
EPS = 1e-5

NC = 2
NS = 16
NW = NC * NS
C = 128
CPW = 80
TOT_CH = NW * CPW
E_PAD = TOT_CH * C
N_ACC = 10240
TRASH = 10100
RPT = N_ACC // NS
ZR = 64
DEGW = 128

def _zero_rows(zref, rows, width):
    def row(r, _):
        def col(cc, __):
            zref[r, pl.ds(cc * 16, 16)] = jnp.zeros((16,), jnp.float32)
            return __
        return lax.fori_loop(0, width // 16, col, _)
    lax.fori_loop(0, rows, row, None)


def _deg_sc_body(dst_hbm, out_hbm, idx_v, ones_v, z_v, acc_sh):
    c = lax.axis_index("c")
    s = lax.axis_index("s")
    w = c * NS + s
    _zero_rows(z_v, ZR, DEGW)
    def orow(r, _):
        def ocol(cc, __):
            ones_v[r, pl.ds(cc * 16, 16)] = jnp.full((16,), 1.0, jnp.float32)
            return __
        return lax.fori_loop(0, DEGW // 16, ocol, _)
    lax.fori_loop(0, C, orow, None)
    def zcopy(k, _):
        pltpu.sync_copy(z_v, acc_sh.at[pl.ds(s * RPT + k * ZR, ZR)])
        return _
    lax.fori_loop(0, RPT // ZR, zcopy, None)
    pltpu.sync_copy(dst_hbm.at[pl.ds(w * CPW, CPW)], idx_v)
    plsc.subcore_barrier()
    def step(j, _):
        pltpu.sync_copy(ones_v, acc_sh.at[idx_v.at[j]], add=True)
        return _
    lax.fori_loop(0, CPW, step, None)
    plsc.subcore_barrier()
    pltpu.sync_copy(acc_sh.at[pl.ds(s * RPT, RPT)],
                    out_hbm.at[c, pl.ds(s * RPT, RPT)])


BLK = 8


def _agg_sc_body(hws_hbm, src_hbm, dst_hbm, out_hbm,
                 src_v, dblk, gbuf0, gbuf1, z_v, acc_sh,
                 gsem0, gsem1, isem):
    c = lax.axis_index("c")
    s = lax.axis_index("s")
    w = c * NS + s
    _zero_rows(z_v, ZR // 2, D)
    def zcopy(k, _):
        pltpu.sync_copy(z_v, acc_sh.at[pl.ds(s * RPT + k * (ZR // 2), ZR // 2)])
        return _
    lax.fori_loop(0, RPT // (ZR // 2), zcopy, None)
    pltpu.sync_copy(src_hbm.at[pl.ds(w * CPW, CPW)], src_v)
    plsc.subcore_barrier()

    gb = (gbuf0, gbuf1)
    gsem = (gsem0, gsem1)

    pltpu.async_copy(hws_hbm.at[src_v.at[0]], gb[0], gsem[0])
    pltpu.async_copy(hws_hbm.at[src_v.at[1]], gb[1], gsem[1])

    @pl.loop(0, CPW, step=BLK)
    def _block(jj):
        pltpu.sync_copy(dst_hbm.at[pl.ds(w * CPW + jj, BLK)], dblk)
        for i in range(BLK):
            j = jj + i
            p = i & 1
            pltpu.make_async_copy(hws_hbm.at[src_v.at[j]],
                                  gb[p], gsem[p]).wait()
            pltpu.sync_copy(gb[p], acc_sh.at[dblk.at[i]], add=True)
            @pl.when(j + 2 < CPW)
            def _refill():
                pltpu.async_copy(hws_hbm.at[src_v.at[j + 2]], gb[p], gsem[p])

    plsc.subcore_barrier()
    pltpu.sync_copy(acc_sh.at[pl.ds(s * RPT, RPT)],
                    out_hbm.at[c, pl.ds(s * RPT, RPT)])


def _mm_body(x_ref, w_ref, o_ref):
    o_ref[...] = jnp.dot(x_ref[...], w_ref[...],
                         preferred_element_type=jnp.float32)


def _scale_body(hw_ref, d0_ref, d1_ref, dinv_ref, hws_ref):
    deg = d0_ref[...] + d1_ref[...] + 1.0
    dinv = lax.rsqrt(deg)
    dinv_ref[...] = dinv
    hws_ref[...] = hw_ref[...] * dinv


def _bn_relu(t, gamma, beta):
    mu = jnp.mean(t, axis=0, keepdims=True)
    var = jnp.mean((t - mu) ** 2, axis=0, keepdims=True)
    y = gamma * (t - mu) * lax.rsqrt(var + EPS) + beta
    return jnp.maximum(y, 0.0)


def _mid_body(a0_ref, a1_ref, hws_ref, dinv_ref, b_ref, g_ref, be_ref,
              w2_ref, hws2_ref):
    t = dinv_ref[...] * (a0_ref[...] + a1_ref[...] + hws_ref[...]) + b_ref[...]
    h1 = _bn_relu(t, g_ref[...], be_ref[...])
    hw2 = jnp.dot(h1, w2_ref[...], preferred_element_type=jnp.float32)
    hws2_ref[...] = hw2 * dinv_ref[...]


def _final_body(a0_ref, a1_ref, hws_ref, dinv_ref, b_ref, g_ref, be_ref,
                x_ref, o_ref):
    t = dinv_ref[...] * (a0_ref[...] + a1_ref[...] + hws_ref[...]) + b_ref[...]
    y = _bn_relu(t, g_ref[...], be_ref[...])
    o_ref[...] = (y + x_ref[...]) * (1.0 / math.sqrt(2.0))


def _tc(body, out_shapes, *args):
    return pl.pallas_call(body, out_shape=out_shapes)(*args)


@functools.cache
def _sc_kernels():
    mesh = plsc.VectorSubcoreMesh(core_axis_name="c", subcore_axis_name="s",
                                  num_cores=NC, num_subcores=NS)
    deg = pl.kernel(
        _deg_sc_body,
        out_type=jax.ShapeDtypeStruct((NC, N_ACC, DEGW), jnp.float32),
        mesh=mesh,
        scratch_types=[
            pltpu.VMEM((CPW, C), jnp.int32),
            pltpu.VMEM((C, DEGW), jnp.float32),
            pltpu.VMEM((ZR, DEGW), jnp.float32),
            pltpu.VMEM_SHARED((N_ACC, DEGW), jnp.float32),
        ],
    )
    agg = pl.kernel(
        _agg_sc_body,
        out_type=jax.ShapeDtypeStruct((NC, N_ACC, D), jnp.float32),
        mesh=mesh,
        scratch_types=[
            pltpu.VMEM((CPW, C), jnp.int32),
            pltpu.VMEM((BLK, C), jnp.int32),
            pltpu.VMEM((C, D), jnp.float32),
            pltpu.VMEM((C, D), jnp.float32),
            pltpu.VMEM((ZR // 2, D), jnp.float32),
            pltpu.VMEM_SHARED((N_ACC, D), jnp.float32),
            pltpu.SemaphoreType.DMA,
            pltpu.SemaphoreType.DMA,
            pltpu.SemaphoreType.DMA,
        ],
    )
    return deg, agg


def kernel(x, edge_index, W1, b1, bn1_gamma, bn1_beta,
           W2, b2, bn2_gamma, bn2_beta):
    src = edge_index[0].astype(jnp.int32)
    dst = edge_index[1].astype(jnp.int32)
    pad = E_PAD - E
    src2d = jnp.concatenate(
        [src, jnp.zeros((pad,), jnp.int32)]).reshape(TOT_CH, C)
    dst2d = jnp.concatenate(
        [dst, jnp.full((pad,), TRASH, jnp.int32)]).reshape(TOT_CH, C)

    _deg_sc, _agg_sc = _sc_kernels()
    degp = _deg_sc(dst2d)
    hw1 = _tc(_mm_body, jax.ShapeDtypeStruct((N, D), jnp.float32), x, W1)

    d0 = degp[0, :N, 0:1]
    d1 = degp[1, :N, 0:1]
    dinv, hws1 = _tc(
        _scale_body,
        (jax.ShapeDtypeStruct((N, 1), jnp.float32),
         jax.ShapeDtypeStruct((N, D), jnp.float32)),
        hw1, d0, d1)

    b1r = b1.reshape(1, D)
    g1r = bn1_gamma.reshape(1, D)
    be1r = bn1_beta.reshape(1, D)
    b2r = b2.reshape(1, D)
    g2r = bn2_gamma.reshape(1, D)
    be2r = bn2_beta.reshape(1, D)

    agg1 = _agg_sc(hws1, src2d, dst2d)
    hws2 = _tc(
        _mid_body, jax.ShapeDtypeStruct((N, D), jnp.float32),
        agg1[0, :N], agg1[1, :N], hws1, dinv, b1r, g1r, be1r, W2)

    agg2 = _agg_sc(hws2, src2d, dst2d)
    out = _tc(
        _final_body, jax.ShapeDtypeStruct((N, D), jnp.float32),
        agg2[0, :N], agg2[1, :N], hws2, dinv, b2r, g2r, be2r, x)
    return out


# speedup vs baseline: 8.7871x; 1.1505x over previous
import functools
import math

import jax
import jax.numpy as jnp
from jax import lax
from jax.experimental import pallas as pl
from jax.experimental.pallas import tpu as pltpu
from jax.experimental.pallas import tpu_sc as plsc

N = 10000
E = 320000
D = 128
EPS = 1e-5

NC = 2
NS = 16
NW = NC * NS
C = 128
CPW = 80
TOT_CH = NW * CPW
E_PAD = TOT_CH * C
N_ACC = 10240
TRASH = 10100
RPT = N_ACC // NS
ZR = 64
DEGW = 128

def _zero_rows(zref, rows, width):
    def row(r, _):
        def col(cc, __):
            zref[r, pl.ds(cc * 16, 16)] = jnp.zeros((16,), jnp.float32)
            return __
        return lax.fori_loop(0, width // 16, col, _)
    lax.fori_loop(0, rows, row, None)


def _deg_sc_body(dst_hbm, out_hbm, idx_v, ones_v, z_v, acc_sh, ssem):
    c = lax.axis_index("c")
    s = lax.axis_index("s")
    w = c * NS + s
    _zero_rows(z_v, ZR, DEGW)
    def orow(r, _):
        def ocol(cc, __):
            ones_v[r, pl.ds(cc * 16, 16)] = jnp.full((16,), 1.0, jnp.float32)
            return __
        return lax.fori_loop(0, DEGW // 16, ocol, _)
    lax.fori_loop(0, C, orow, None)
    def zcopy(k, _):
        pltpu.sync_copy(z_v, acc_sh.at[pl.ds(s * RPT + k * ZR, ZR)])
        return _
    lax.fori_loop(0, RPT // ZR, zcopy, None)
    pltpu.sync_copy(dst_hbm.at[pl.ds(w * CPW, CPW)], idx_v)
    plsc.subcore_barrier()

    @pl.loop(0, CPW, step=BLK)
    def _block(jj):
        cps = [pltpu.async_copy(ones_v, acc_sh.at[idx_v.at[jj + i]],
                                ssem, add=True)
               for i in range(BLK)]
        for cp in cps:
            cp.wait()

    plsc.subcore_barrier()
    pltpu.sync_copy(acc_sh.at[pl.ds(s * RPT, RPT)],
                    out_hbm.at[c, pl.ds(s * RPT, RPT)])


BLK = 8


def _agg_sc_body(hws_hbm, src_hbm, dst_hbm, out_hbm,
                 src_v, dblk, gbuf0, gbuf1, z_v, acc_sh,
                 gsem0, gsem1, isem):
    c = lax.axis_index("c")
    s = lax.axis_index("s")
    w = c * NS + s
    _zero_rows(z_v, ZR // 2, D)
    def zcopy(k, _):
        pltpu.sync_copy(z_v, acc_sh.at[pl.ds(s * RPT + k * (ZR // 2), ZR // 2)])
        return _
    lax.fori_loop(0, RPT // (ZR // 2), zcopy, None)
    pltpu.sync_copy(src_hbm.at[pl.ds(w * CPW, CPW)], src_v)
    plsc.subcore_barrier()

    gb = (gbuf0, gbuf1)
    gsem = (gsem0, gsem1)

    pltpu.async_copy(hws_hbm.at[src_v.at[0]], gb[0], gsem[0])
    pltpu.async_copy(hws_hbm.at[src_v.at[1]], gb[1], gsem[1])

    @pl.loop(0, CPW, step=BLK)
    def _block(jj):
        pltpu.sync_copy(dst_hbm.at[pl.ds(w * CPW + jj, BLK)], dblk)
        for i in range(BLK):
            j = jj + i
            p = i & 1
            pltpu.make_async_copy(hws_hbm.at[src_v.at[j]],
                                  gb[p], gsem[p]).wait()
            pltpu.sync_copy(gb[p], acc_sh.at[dblk.at[i]], add=True)
            @pl.when(j + 2 < CPW)
            def _refill():
                pltpu.async_copy(hws_hbm.at[src_v.at[j + 2]], gb[p], gsem[p])

    plsc.subcore_barrier()
    pltpu.sync_copy(acc_sh.at[pl.ds(s * RPT, RPT)],
                    out_hbm.at[c, pl.ds(s * RPT, RPT)])


def _mm_body(x_ref, w_ref, o_ref):
    o_ref[...] = jnp.dot(x_ref[...], w_ref[...],
                         preferred_element_type=jnp.float32)


def _scale_body(hw_ref, d0_ref, d1_ref, dinv_ref, hws_ref):
    deg = d0_ref[...] + d1_ref[...] + 1.0
    dinv = lax.rsqrt(deg)
    dinv_ref[...] = dinv
    hws_ref[...] = hw_ref[...] * dinv


def _bn_relu(t, gamma, beta):
    mu = jnp.mean(t, axis=0, keepdims=True)
    var = jnp.mean((t - mu) ** 2, axis=0, keepdims=True)
    y = gamma * (t - mu) * lax.rsqrt(var + EPS) + beta
    return jnp.maximum(y, 0.0)


def _mid_body(aggp_ref, hws_ref, dinv_ref, b_ref, g_ref, be_ref,
              w2_ref, hws2_ref):
    a = aggp_ref[0, :N, :] + aggp_ref[1, :N, :]
    t = dinv_ref[...] * (a + hws_ref[...]) + b_ref[...]
    h1 = _bn_relu(t, g_ref[...], be_ref[...])
    hw2 = jnp.dot(h1, w2_ref[...], preferred_element_type=jnp.float32)
    hws2_ref[...] = hw2 * dinv_ref[...]


def _final_body(aggp_ref, hws_ref, dinv_ref, b_ref, g_ref, be_ref,
                x_ref, o_ref):
    a = aggp_ref[0, :N, :] + aggp_ref[1, :N, :]
    t = dinv_ref[...] * (a + hws_ref[...]) + b_ref[...]
    y = _bn_relu(t, g_ref[...], be_ref[...])
    o_ref[...] = (y + x_ref[...]) * (1.0 / math.sqrt(2.0))


def _tc(body, out_shapes, *args):
    return pl.pallas_call(body, out_shape=out_shapes)(*args)


@functools.cache
def _sc_kernels():
    mesh = plsc.VectorSubcoreMesh(core_axis_name="c", subcore_axis_name="s",
                                  num_cores=NC, num_subcores=NS)
    deg = pl.kernel(
        _deg_sc_body,
        out_type=jax.ShapeDtypeStruct((NC, N_ACC, DEGW), jnp.float32),
        mesh=mesh,
        scratch_types=[
            pltpu.VMEM((CPW, C), jnp.int32),
            pltpu.VMEM((C, DEGW), jnp.float32),
            pltpu.VMEM((ZR, DEGW), jnp.float32),
            pltpu.VMEM_SHARED((N_ACC, DEGW), jnp.float32),
            pltpu.SemaphoreType.DMA,
        ],
    )
    agg = pl.kernel(
        _agg_sc_body,
        out_type=jax.ShapeDtypeStruct((NC, N_ACC, D), jnp.float32),
        mesh=mesh,
        scratch_types=[
            pltpu.VMEM((CPW, C), jnp.int32),
            pltpu.VMEM((BLK, C), jnp.int32),
            pltpu.VMEM((C, D), jnp.float32),
            pltpu.VMEM((C, D), jnp.float32),
            pltpu.VMEM((ZR // 2, D), jnp.float32),
            pltpu.VMEM_SHARED((N_ACC, D), jnp.float32),
            pltpu.SemaphoreType.DMA,
            pltpu.SemaphoreType.DMA,
            pltpu.SemaphoreType.DMA,
        ],
    )
    return deg, agg


def kernel(x, edge_index, W1, b1, bn1_gamma, bn1_beta,
           W2, b2, bn2_gamma, bn2_beta):
    src = edge_index[0].astype(jnp.int32)
    dst = edge_index[1].astype(jnp.int32)
    pad = E_PAD - E
    src2d = jnp.concatenate(
        [src, jnp.zeros((pad,), jnp.int32)]).reshape(TOT_CH, C)
    dst2d = jnp.concatenate(
        [dst, jnp.full((pad,), TRASH, jnp.int32)]).reshape(TOT_CH, C)

    _deg_sc, _agg_sc = _sc_kernels()
    degp = _deg_sc(dst2d)
    hw1 = _tc(_mm_body, jax.ShapeDtypeStruct((N, D), jnp.float32), x, W1)

    d0 = degp[0, :N, 0:1]
    d1 = degp[1, :N, 0:1]
    dinv, hws1 = _tc(
        _scale_body,
        (jax.ShapeDtypeStruct((N, 1), jnp.float32),
         jax.ShapeDtypeStruct((N, D), jnp.float32)),
        hw1, d0, d1)

    b1r = b1.reshape(1, D)
    g1r = bn1_gamma.reshape(1, D)
    be1r = bn1_beta.reshape(1, D)
    b2r = b2.reshape(1, D)
    g2r = bn2_gamma.reshape(1, D)
    be2r = bn2_beta.reshape(1, D)

    agg1 = _agg_sc(hws1, src2d, dst2d)
    hws2 = _tc(
        _mid_body, jax.ShapeDtypeStruct((N, D), jnp.float32),
        agg1, hws1, dinv, b1r, g1r, be1r, W2)

    agg2 = _agg_sc(hws2, src2d, dst2d)
    out = _tc(
        _final_body, jax.ShapeDtypeStruct((N, D), jnp.float32),
        agg2, hws2, dinv, b2r, g2r, be2r, x)
    return out
